# Initial kernel scaffold; baseline (speedup 1.0000x reference)
import functools

import jax
import jax.numpy as jnp
from jax import lax
from jax.experimental import pallas as pl
from jax.experimental.pallas import tpu as pltpu
from jax.experimental.pallas import tpu_sc as plsc

NC = 2
NS = 16
CG = 4
CHUNK = 2048


def _prep_kernel(im0_ref, flow_ref, idx_ref, sval_ref, *, hblk, W, H):
    hb = pl.program_id(1)
    n = hblk * W
    fx = flow_ref[0, :, :, 0]
    fy = flow_ref[0, :, :, 1]
    gx = lax.broadcasted_iota(jnp.float32, (hblk, W), 1)
    gy = lax.broadcasted_iota(jnp.float32, (hblk, W), 0) + (hb * hblk).astype(
        jnp.float32
    )
    x = gx + fx
    y = gy + fy
    x0 = jnp.floor(x)
    y0 = jnp.floor(y)
    frx = x - x0
    fry = y - y0

    v = im0_ref[0].reshape(8, n)
    vt = jnp.transpose(v, (1, 0))

    for h in (0, 1):
        iy = y0 + h
        wy = fry if h else (1.0 - fry)
        iyi = jnp.clip(iy.astype(jnp.int32), 0, H - 1)
        yok = (iy >= 0) & (iy < H)
        idx_parts = []
        w_parts = []
        for s in (0, 1):
            ix = x0 + s
            wx = frx if s else (1.0 - frx)
            ixi = jnp.clip(ix.astype(jnp.int32), 0, W - 1)
            ok = yok & (ix >= 0) & (ix < W)
            w = jnp.where(ok, wx * wy, 0.0)
            idx_parts.append((iyi * W + ixi).reshape(n, 1))
            w_parts.append(w.reshape(n, 1))
        idx2 = jnp.concatenate(idx_parts, axis=1)
        w2 = jnp.concatenate(w_parts, axis=1)
        idx_ref[0, h] = idx2.reshape(n * 2 // 128, 128)
        for g in (0, 1):
            sv = w2[:, :, None] * vt[:, None, g * CG:(g + 1) * CG]
            sval_ref[g, 0, h] = sv.reshape(n * 2, CG)


def _post_kernel(acc_ref, out_ref, *, hblk, W):
    n = hblk * W
    a = jnp.concatenate([acc_ref[0, 0], acc_ref[1, 0]], axis=1)
    out_ref[0] = jnp.transpose(a, (1, 0)).reshape(8, hblk, W)


def _sc_scatter(sval_hbm, idx_hbm, zeros_hbm, out_hbm, acc, vbuf, ibuf,
                sem_v, sem_i, *, B, HW):
    core = lax.axis_index("c")
    sid = lax.axis_index("s")
    slc = HW // NS
    recs_h = 2 * HW
    t_recs = recs_h // NS
    nchunk = t_recs // CHUNK
    nrounds = B * 2

    @pl.loop(core, nrounds, step=NC)
    def _round(r):
        b = r // 2
        g = lax.rem(r, 2)
        pltpu.sync_copy(zeros_hbm, acc.at[pl.ds(sid * slc, slc)])
        plsc.subcore_barrier()

        @pl.loop(0, 2)
        def _h(h):
            @pl.loop(0, nchunk)
            def _chunk(k):
                rec0 = sid * t_recs + k * CHUNK
                row0 = rec0 // 128
                cp_v = pltpu.async_copy(
                    sval_hbm.at[g, b, h, pl.ds(rec0, CHUNK)], vbuf, sem_v)
                cp_i = pltpu.async_copy(
                    idx_hbm.at[b, h, pl.ds(row0, CHUNK // 128)], ibuf, sem_i)
                cp_v.wait()
                cp_i.wait()
                for j in range(CHUNK // 128):
                    pltpu.sync_copy(
                        vbuf.at[pl.ds(j * 128, 128)],
                        acc.at[ibuf.at[j]],
                        add=True,
                    )

        plsc.subcore_barrier()
        pltpu.sync_copy(
            acc.at[pl.ds(sid * slc, slc)],
            out_hbm.at[g, b, pl.ds(sid * slc, slc)],
        )
        plsc.subcore_barrier()


def kernel(im0, flow):
    B, C, H, W = im0.shape
    HW = H * W
    hblk = 64
    n = hblk * W

    grid = (B, H // hblk)
    idx, sval = pl.pallas_call(
        functools.partial(_prep_kernel, hblk=hblk, W=W, H=H),
        grid=grid,
        in_specs=[
            pl.BlockSpec((1, 8, hblk, W), lambda b, hb: (b, 0, hb, 0)),
            pl.BlockSpec((1, hblk, W, 2), lambda b, hb: (b, hb, 0, 0)),
        ],
        out_specs=[
            pl.BlockSpec((1, 2, n * 2 // 128, 128),
                         lambda b, hb: (b, 0, hb, 0)),
            pl.BlockSpec((2, 1, 2, n * 2, 4), lambda b, hb: (0, b, 0, hb, 0)),
        ],
        out_shape=[
            jax.ShapeDtypeStruct((B, 2, HW * 2 // 128, 128), jnp.int32),
            jax.ShapeDtypeStruct((2, B, 2, HW * 2, 4), jnp.float32),
        ],
    )(im0, flow)

    zeros = jnp.zeros((HW // NS, CG), jnp.float32)

    mesh = plsc.VectorSubcoreMesh(core_axis_name="c", subcore_axis_name="s")
    acc = pl.kernel(
        functools.partial(_sc_scatter, B=B, HW=HW),
        out_type=jax.ShapeDtypeStruct((2, B, HW, CG), jnp.float32),
        mesh=mesh,
        scratch_types=[
            pltpu.VMEM_SHARED((HW, CG), jnp.float32),
            pltpu.VMEM((CHUNK, CG), jnp.float32),
            pltpu.VMEM((CHUNK // 128, 128), jnp.int32),
            pltpu.SemaphoreType.DMA,
            pltpu.SemaphoreType.DMA,
        ],
    )(sval, idx, zeros)

    out = pl.pallas_call(
        functools.partial(_post_kernel, hblk=hblk, W=W),
        grid=grid,
        in_specs=[
            pl.BlockSpec((2, 1, n, 4), lambda b, hb: (0, b, hb, 0)),
        ],
        out_specs=pl.BlockSpec((1, 8, hblk, W), lambda b, hb: (b, 0, hb, 0)),
        out_shape=jax.ShapeDtypeStruct((B, C, H, W), jnp.float32),
    )(acc)
    return out


# speedup vs baseline: 2.4403x; 2.4403x over previous
import functools

import jax
import jax.numpy as jnp
from jax import lax
from jax.experimental import pallas as pl
from jax.experimental.pallas import tpu as pltpu
from jax.experimental.pallas import tpu_sc as plsc

NC = 2
NS = 16
DUMP = 2048
CHUNK = 2048


def _prep_kernel(im0_ref, flow_ref, idx_ref, svp_ref, *, hblk, W, H):
    hb = pl.program_id(1)
    half = (H // 2) * W
    fx = flow_ref[0, :, :, 0]
    fy = flow_ref[0, :, :, 1]
    gxi = lax.broadcasted_iota(jnp.int32, (hblk, W), 1)
    gyi = lax.broadcasted_iota(jnp.int32, (hblk, W), 0) + hb * hblk
    x = gxi.astype(jnp.float32) + fx
    y = gyi.astype(jnp.float32) + fy
    x0 = jnp.floor(x)
    y0 = jnp.floor(y)
    frx = x - x0
    fry = y - y0
    p = gyi * W + gxi
    dump = half + jnp.bitwise_and(p, DUMP - 1)

    k = 0
    for h in (0, 1):
        iy = y0 + h
        wy = fry if h else (1.0 - fry)
        iyi = jnp.clip(iy.astype(jnp.int32), 0, H - 1)
        yok = (iy >= 0) & (iy < H)
        for s in (0, 1):
            ix = x0 + s
            wx = frx if s else (1.0 - frx)
            ixi = jnp.clip(ix.astype(jnp.int32), 0, W - 1)
            ok = yok & (ix >= 0) & (ix < W)
            w = jnp.where(ok, wx * wy, 0.0)
            gidx = iyi * W + ixi
            idx_ref[0, 0, k] = jnp.where(gidx < half, gidx, dump)
            idx_ref[1, 0, k] = jnp.where(gidx >= half, gidx - half, dump)
            for c in range(8):
                svp_ref[0, k, c] = w * im0_ref[0, c]
            k += 1


def _post_kernel(acc_ref, out_ref, *, hblk, W):
    out_ref[0] = jnp.transpose(acc_ref[0], (1, 0)).reshape(8, hblk, W)


def _sc_scatter(sval_hbm, idx_hbm, zeros_hbm, out_hbm, acc, vbuf, ibuf,
                sem_v, sem_i, *, B, HW):
    core = lax.axis_index("c")
    sid = lax.axis_index("s")
    half = HW // 2
    slc = half // NS
    t_recs = HW // NS
    nchunk = t_recs // CHUNK
    my0 = pl.multiple_of(sid * slc, slc)

    @pl.loop(0, B)
    def _round(b):
        pltpu.sync_copy(zeros_hbm, acc.at[pl.ds(my0, slc)])
        plsc.subcore_barrier()

        @pl.loop(0, 4)
        def _corner(k):
            @pl.loop(0, nchunk)
            def _chunk(q):
                rec0 = pl.multiple_of(sid * t_recs + q * CHUNK, CHUNK)
                row0 = pl.multiple_of(rec0 // 128, CHUNK // 128)
                cp_v = pltpu.async_copy(
                    sval_hbm.at[b, k, pl.ds(rec0, CHUNK)], vbuf, sem_v)
                cp_i = pltpu.async_copy(
                    idx_hbm.at[core, b, k, pl.ds(row0, CHUNK // 128)],
                    ibuf, sem_i)
                cp_v.wait()
                cp_i.wait()
                for j in range(CHUNK // 128):
                    pltpu.sync_copy(
                        vbuf.at[pl.ds(j * 128, 128)],
                        acc.at[ibuf.at[j]],
                        add=True,
                    )

        plsc.subcore_barrier()
        out0 = pl.multiple_of(core * half + sid * slc, slc)
        pltpu.sync_copy(
            acc.at[pl.ds(my0, slc)],
            out_hbm.at[b, pl.ds(out0, slc)],
        )
        plsc.subcore_barrier()


def kernel(im0, flow):
    B, C, H, W = im0.shape
    HW = H * W
    half = HW // 2
    hblk = 32
    n = hblk * W

    grid = (B, H // hblk)
    idx, svp = pl.pallas_call(
        functools.partial(_prep_kernel, hblk=hblk, W=W, H=H),
        grid=grid,
        in_specs=[
            pl.BlockSpec((1, 8, hblk, W), lambda b, hb: (b, 0, hb, 0)),
            pl.BlockSpec((1, hblk, W, 2), lambda b, hb: (b, hb, 0, 0)),
        ],
        out_specs=[
            pl.BlockSpec((2, 1, 4, hblk, W), lambda b, hb: (0, b, 0, hb, 0)),
            pl.BlockSpec((1, 4, 8, hblk, W), lambda b, hb: (b, 0, 0, hb, 0)),
        ],
        out_shape=[
            jax.ShapeDtypeStruct((2, B, 4, H, W), jnp.int32),
            jax.ShapeDtypeStruct((B, 4, 8, H, W), jnp.float32),
        ],
    )(im0, flow)

    sval = jnp.transpose(svp.reshape(B, 4, 8, HW), (0, 1, 3, 2))
    idx = idx.reshape(2, B, 4, HW // 128, 128)
    zeros = jnp.zeros((half // NS, 8), jnp.float32)

    mesh = plsc.VectorSubcoreMesh(core_axis_name="c", subcore_axis_name="s")
    acc = pl.kernel(
        functools.partial(_sc_scatter, B=B, HW=HW),
        out_type=jax.ShapeDtypeStruct((B, HW, 8), jnp.float32),
        mesh=mesh,
        compiler_params=pltpu.CompilerParams(use_tc_tiling_on_sc=False),
        scratch_types=[
            pltpu.VMEM_SHARED((half + DUMP, 8), jnp.float32),
            pltpu.VMEM((CHUNK, 8), jnp.float32),
            pltpu.VMEM((CHUNK // 128, 128), jnp.int32),
            pltpu.SemaphoreType.DMA,
            pltpu.SemaphoreType.DMA,
        ],
    )(sval, idx, zeros)

    out = pl.pallas_call(
        functools.partial(_post_kernel, hblk=hblk, W=W),
        grid=grid,
        in_specs=[
            pl.BlockSpec((1, n, 8), lambda b, hb: (b, hb, 0)),
        ],
        out_specs=pl.BlockSpec((1, 8, hblk, W), lambda b, hb: (b, 0, hb, 0)),
        out_shape=jax.ShapeDtypeStruct((B, C, H, W), jnp.float32),
    )(acc)
    return out


# speedup vs baseline: 3.8561x; 1.5802x over previous
import functools

import jax
import jax.numpy as jnp
from jax import lax
from jax.experimental import pallas as pl
from jax.experimental.pallas import tpu as pltpu
from jax.experimental.pallas import tpu_sc as plsc

NC = 2
NS = 16
DUMP = 2048
PIX = 512


def _prep_kernel(im0_ref, flow_ref, idx_ref, wts_ref, *, hblk, W, H):
    del im0_ref
    hb = pl.program_id(1)
    half = (H // 2) * W
    fx = flow_ref[0, :, :, 0]
    fy = flow_ref[0, :, :, 1]
    gxi = lax.broadcasted_iota(jnp.int32, (hblk, W), 1)
    gyi = lax.broadcasted_iota(jnp.int32, (hblk, W), 0) + hb * hblk
    x = gxi.astype(jnp.float32) + fx
    y = gyi.astype(jnp.float32) + fy
    x0 = jnp.floor(x)
    y0 = jnp.floor(y)
    frx = x - x0
    fry = y - y0
    p = gyi * W + gxi
    dump = half + jnp.bitwise_and(p, DUMP - 1)

    k = 0
    for h in (0, 1):
        iy = y0 + h
        wy = fry if h else (1.0 - fry)
        iyi = jnp.clip(iy.astype(jnp.int32), 0, H - 1)
        yok = (iy >= 0) & (iy < H)
        for s in (0, 1):
            ix = x0 + s
            wx = frx if s else (1.0 - frx)
            ixi = jnp.clip(ix.astype(jnp.int32), 0, W - 1)
            ok = yok & (ix >= 0) & (ix < W)
            wts_ref[0, k] = jnp.where(ok, wx * wy, 0.0)
            gidx = iyi * W + ixi
            idx_ref[0, 0, k] = jnp.where(gidx < half, gidx, dump)
            idx_ref[1, 0, k] = jnp.where(gidx >= half, gidx - half, dump)
            k += 1


def _post_kernel(acc_ref, out_ref, *, hblk, W):
    out_ref[0] = jnp.transpose(acc_ref[0], (1, 0)).reshape(8, hblk, W)


def _sc_scatter(im_hbm, wts_hbm, idx_hbm, zeros_hbm, out_hbm, acc, imbuf,
                wbuf, ibuf, stage, sem_in, *, B, HW):
    core = lax.axis_index("c")
    sid = lax.axis_index("s")
    half = HW // 2
    slc = half // NS
    t_pix = HW // NS
    nchunk = t_pix // PIX
    my0 = pl.multiple_of(sid * slc, slc)
    iota = lax.iota(jnp.int32, 16)
    cols = [jnp.full((16,), c, jnp.int32) for c in range(8)]

    @pl.loop(0, B)
    def _round(b):
        pltpu.sync_copy(zeros_hbm, acc.at[pl.ds(my0, slc)])
        plsc.subcore_barrier()

        @pl.loop(0, nchunk)
        def _chunk(q):
            p0 = pl.multiple_of(sid * t_pix + q * PIX, PIX)
            r0 = pl.multiple_of(p0 // 128, PIX // 128)
            cp_m = pltpu.async_copy(
                im_hbm.at[b, :, pl.ds(p0, PIX)], imbuf, sem_in)
            cp_w = pltpu.async_copy(
                wts_hbm.at[b, :, pl.ds(p0, PIX)], wbuf, sem_in)
            cp_i = pltpu.async_copy(
                idx_hbm.at[core, b, :, pl.ds(r0, PIX // 128)], ibuf, sem_in)
            cp_m.wait()
            cp_w.wait()
            cp_i.wait()
            for k in range(4):
                for g in range(PIX // 16):
                    rows = iota + (k * PIX + g * 16)
                    wv = wbuf[k, pl.ds(g * 16, 16)]
                    for c in range(8):
                        prod = imbuf[c, pl.ds(g * 16, 16)] * wv
                        plsc.store_scatter(stage, [rows, cols[c]], prod)
            for k in range(4):
                for j in range(PIX // 128):
                    pltpu.sync_copy(
                        stage.at[pl.ds(k * PIX + j * 128, 128)],
                        acc.at[ibuf.at[k, j]],
                        add=True,
                    )

        plsc.subcore_barrier()
        out0 = pl.multiple_of(core * half + sid * slc, slc)
        pltpu.sync_copy(
            acc.at[pl.ds(my0, slc)],
            out_hbm.at[b, pl.ds(out0, slc)],
        )
        plsc.subcore_barrier()


def kernel(im0, flow):
    B, C, H, W = im0.shape
    HW = H * W
    half = HW // 2
    hblk = 32
    n = hblk * W

    grid = (B, H // hblk)
    idx, wts = pl.pallas_call(
        functools.partial(_prep_kernel, hblk=hblk, W=W, H=H),
        grid=grid,
        in_specs=[
            pl.BlockSpec((1, 8, hblk, W), lambda b, hb: (b, 0, hb, 0)),
            pl.BlockSpec((1, hblk, W, 2), lambda b, hb: (b, hb, 0, 0)),
        ],
        out_specs=[
            pl.BlockSpec((2, 1, 4, hblk, W), lambda b, hb: (0, b, 0, hb, 0)),
            pl.BlockSpec((1, 4, hblk, W), lambda b, hb: (b, 0, hb, 0)),
        ],
        out_shape=[
            jax.ShapeDtypeStruct((2, B, 4, H, W), jnp.int32),
            jax.ShapeDtypeStruct((B, 4, H, W), jnp.float32),
        ],
    )(im0, flow)

    idx = idx.reshape(2, B, 4, HW // 128, 128)
    wtsf = wts.reshape(B, 4, HW)
    im0f = im0.reshape(B, 8, HW)
    zeros = jnp.zeros((half // NS, 8), jnp.float32)

    mesh = plsc.VectorSubcoreMesh(core_axis_name="c", subcore_axis_name="s")
    acc = pl.kernel(
        functools.partial(_sc_scatter, B=B, HW=HW),
        out_type=jax.ShapeDtypeStruct((B, HW, 8), jnp.float32),
        mesh=mesh,
        compiler_params=pltpu.CompilerParams(
            use_tc_tiling_on_sc=False, needs_layout_passes=False
        ),
        scratch_types=[
            pltpu.VMEM_SHARED((half + DUMP, 8), jnp.float32),
            pltpu.VMEM((8, PIX), jnp.float32),
            pltpu.VMEM((4, PIX), jnp.float32),
            pltpu.VMEM((4, PIX // 128, 128), jnp.int32),
            pltpu.VMEM((4 * PIX, 8), jnp.float32),
            pltpu.SemaphoreType.DMA,
        ],
    )(im0f, wtsf, idx, zeros)

    out = pl.pallas_call(
        functools.partial(_post_kernel, hblk=hblk, W=W),
        grid=grid,
        in_specs=[
            pl.BlockSpec((1, n, 8), lambda b, hb: (b, hb, 0)),
        ],
        out_specs=pl.BlockSpec((1, 8, hblk, W), lambda b, hb: (b, 0, hb, 0)),
        out_shape=jax.ShapeDtypeStruct((B, C, H, W), jnp.float32),
    )(acc)
    return out


# speedup vs baseline: 3.8800x; 1.0062x over previous
import functools

import jax
import jax.numpy as jnp
from jax import lax
from jax.experimental import pallas as pl
from jax.experimental.pallas import tpu as pltpu
from jax.experimental.pallas import tpu_sc as plsc

NC = 2
NS = 16
DUMP = 2048
PIX = 512


def _prep_kernel(im0_ref, flow_ref, idx_ref, wts_ref, *, hblk, W, H):
    del im0_ref
    hb = pl.program_id(1)
    half = (H // 2) * W
    fx = flow_ref[0, :, :, 0]
    fy = flow_ref[0, :, :, 1]
    gxi = lax.broadcasted_iota(jnp.int32, (hblk, W), 1)
    gyi = lax.broadcasted_iota(jnp.int32, (hblk, W), 0) + hb * hblk
    x = gxi.astype(jnp.float32) + fx
    y = gyi.astype(jnp.float32) + fy
    x0 = jnp.floor(x)
    y0 = jnp.floor(y)
    frx = x - x0
    fry = y - y0
    p = gyi * W + gxi
    dump = half + jnp.bitwise_and(p, DUMP - 1)

    k = 0
    for h in (0, 1):
        iy = y0 + h
        wy = fry if h else (1.0 - fry)
        iyi = jnp.clip(iy.astype(jnp.int32), 0, H - 1)
        yok = (iy >= 0) & (iy < H)
        for s in (0, 1):
            ix = x0 + s
            wx = frx if s else (1.0 - frx)
            ixi = jnp.clip(ix.astype(jnp.int32), 0, W - 1)
            ok = yok & (ix >= 0) & (ix < W)
            wts_ref[0, k] = jnp.where(ok, wx * wy, 0.0)
            gidx = iyi * W + ixi
            idx_ref[0, 0, k] = jnp.where(gidx < half, gidx, dump)
            idx_ref[1, 0, k] = jnp.where(gidx >= half, gidx - half, dump)
            k += 1


def _post_kernel(acc_ref, out_ref, *, hblk, W):
    out_ref[0] = jnp.transpose(acc_ref[0], (1, 0)).reshape(8, hblk, W)


def _sc_scatter(im_hbm, wts_hbm, idx_hbm, zeros_hbm, out_hbm, acc, imbuf,
                wbuf, ibuf, stage, sem_in, *, B, HW):
    core = lax.axis_index("c")
    sid = lax.axis_index("s")
    half = HW // 2
    slc = half // NS
    t_pix = HW // NS
    nchunk = t_pix // PIX
    my0 = pl.multiple_of(sid * slc, slc)
    iota = lax.iota(jnp.int32, 16)
    cols = [jnp.full((16,), c, jnp.int32) for c in range(8)]

    @pl.loop(0, B)
    def _round(b):
        pltpu.sync_copy(zeros_hbm, acc.at[pl.ds(my0, slc)])
        plsc.subcore_barrier()

        @pl.loop(0, nchunk)
        def _chunk(q):
            p0 = pl.multiple_of(sid * t_pix + q * PIX, PIX)
            r0 = pl.multiple_of(p0 // 128, PIX // 128)
            cp_m = pltpu.async_copy(
                im_hbm.at[b, :, pl.ds(p0, PIX)], imbuf, sem_in)
            cp_w = pltpu.async_copy(
                wts_hbm.at[b, :, pl.ds(p0, PIX)], wbuf, sem_in)
            cp_i = pltpu.async_copy(
                idx_hbm.at[core, b, :, pl.ds(r0, PIX // 128)], ibuf, sem_in)
            cp_m.wait()
            cp_w.wait()
            cp_i.wait()
            for k in range(4):
                for g in range(PIX // 16):
                    rows = iota + (k * PIX + g * 16)
                    wv = wbuf[k, pl.ds(g * 16, 16)]
                    for c in range(8):
                        prod = imbuf[c, pl.ds(g * 16, 16)] * wv
                        plsc.store_scatter(stage, [rows, cols[c]], prod)
            for k in range(4):
                for j in range(PIX // 128):
                    pltpu.sync_copy(
                        stage.at[pl.ds(k * PIX + j * 128, 128)],
                        acc.at[ibuf.at[k, j]],
                        add=True,
                    )

        plsc.subcore_barrier()
        out0 = pl.multiple_of(core * half + sid * slc, slc)
        pltpu.sync_copy(
            acc.at[pl.ds(my0, slc)],
            out_hbm.at[b, pl.ds(out0, slc)],
        )
        plsc.subcore_barrier()


def kernel(im0, flow):
    B, C, H, W = im0.shape
    HW = H * W
    half = HW // 2
    hblk = 64
    n = hblk * W

    grid = (B, H // hblk)
    idx, wts = pl.pallas_call(
        functools.partial(_prep_kernel, hblk=hblk, W=W, H=H),
        grid=grid,
        in_specs=[
            pl.BlockSpec((1, 8, hblk, W), lambda b, hb: (b, 0, hb, 0)),
            pl.BlockSpec((1, hblk, W, 2), lambda b, hb: (b, hb, 0, 0)),
        ],
        out_specs=[
            pl.BlockSpec((2, 1, 4, hblk, W), lambda b, hb: (0, b, 0, hb, 0)),
            pl.BlockSpec((1, 4, hblk, W), lambda b, hb: (b, 0, hb, 0)),
        ],
        out_shape=[
            jax.ShapeDtypeStruct((2, B, 4, H, W), jnp.int32),
            jax.ShapeDtypeStruct((B, 4, H, W), jnp.float32),
        ],
    )(im0, flow)

    idx = idx.reshape(2, B, 4, HW // 128, 128)
    wtsf = wts.reshape(B, 4, HW)
    im0f = im0.reshape(B, 8, HW)
    zeros = jnp.zeros((half // NS, 8), jnp.float32)

    mesh = plsc.VectorSubcoreMesh(core_axis_name="c", subcore_axis_name="s")
    acc = pl.kernel(
        functools.partial(_sc_scatter, B=B, HW=HW),
        out_type=jax.ShapeDtypeStruct((B, HW, 8), jnp.float32),
        mesh=mesh,
        compiler_params=pltpu.CompilerParams(
            use_tc_tiling_on_sc=False, needs_layout_passes=False
        ),
        scratch_types=[
            pltpu.VMEM_SHARED((half + DUMP, 8), jnp.float32),
            pltpu.VMEM((8, PIX), jnp.float32),
            pltpu.VMEM((4, PIX), jnp.float32),
            pltpu.VMEM((4, PIX // 128, 128), jnp.int32),
            pltpu.VMEM((4 * PIX, 8), jnp.float32),
            pltpu.SemaphoreType.DMA,
        ],
    )(im0f, wtsf, idx, zeros)

    out = pl.pallas_call(
        functools.partial(_post_kernel, hblk=hblk, W=W),
        grid=grid,
        in_specs=[
            pl.BlockSpec((1, n, 8), lambda b, hb: (b, hb, 0)),
        ],
        out_specs=pl.BlockSpec((1, 8, hblk, W), lambda b, hb: (b, 0, hb, 0)),
        out_shape=jax.ShapeDtypeStruct((B, C, H, W), jnp.float32),
    )(acc)
    return out


# speedup vs baseline: 5.4974x; 1.4168x over previous
import functools

import jax
import jax.numpy as jnp
from jax import lax
from jax.experimental import pallas as pl
from jax.experimental.pallas import tpu as pltpu
from jax.experimental.pallas import tpu_sc as plsc

NC = 2
NS = 16
DUMP = 2048
PIX = 512


def _prep_kernel(im0_ref, flow_ref, idx_ref, wts_ref, imr_ref, *, hblk, W, H):
    hb = pl.program_id(1)
    half = (H // 2) * W
    nr = hblk * W // 128
    fx = flow_ref[0, :, :, 0]
    fy = flow_ref[0, :, :, 1]
    gxi = lax.broadcasted_iota(jnp.int32, (hblk, W), 1)
    gyi = lax.broadcasted_iota(jnp.int32, (hblk, W), 0) + hb * hblk
    x = gxi.astype(jnp.float32) + fx
    y = gyi.astype(jnp.float32) + fy
    x0 = jnp.floor(x)
    y0 = jnp.floor(y)
    frx = x - x0
    fry = y - y0
    p = gyi * W + gxi
    dump = half + jnp.bitwise_and(p, DUMP - 1)

    for c in range(8):
        imr_ref[0, c] = im0_ref[0, c].reshape(nr, 128)

    k = 0
    for h in (0, 1):
        iy = y0 + h
        wy = fry if h else (1.0 - fry)
        iyi = jnp.clip(iy.astype(jnp.int32), 0, H - 1)
        yok = (iy >= 0) & (iy < H)
        for s in (0, 1):
            ix = x0 + s
            wx = frx if s else (1.0 - frx)
            ixi = jnp.clip(ix.astype(jnp.int32), 0, W - 1)
            ok = yok & (ix >= 0) & (ix < W)
            wts_ref[0, k] = jnp.where(ok, wx * wy, 0.0).reshape(nr, 128)
            gidx = iyi * W + ixi
            idx_ref[0, 0, k] = jnp.where(gidx < half, gidx, dump).reshape(
                nr, 128)
            idx_ref[1, 0, k] = jnp.where(gidx >= half, gidx - half,
                                         dump).reshape(nr, 128)
            k += 1


def _post_kernel(acc_ref, out_ref, *, hblk, W):
    out_ref[0] = jnp.transpose(acc_ref[0], (1, 0)).reshape(8, hblk, W)


def _sc_scatter(im_hbm, wts_hbm, idx_hbm, zeros_hbm, out_hbm, acc, imbuf,
                wbuf, ibuf, stage, sem_in, *, B, HW):
    core = lax.axis_index("c")
    sid = lax.axis_index("s")
    half = HW // 2
    slc = half // NS
    t_pix = HW // NS
    nchunk = t_pix // PIX
    my0 = pl.multiple_of(sid * slc, slc)
    iota = lax.iota(jnp.int32, 16)
    cols = [jnp.full((16,), c, jnp.int32) for c in range(8)]

    @pl.loop(0, B)
    def _round(b):
        pltpu.sync_copy(zeros_hbm, acc.at[pl.ds(my0, slc)])
        plsc.subcore_barrier()

        @pl.loop(0, nchunk)
        def _chunk(q):
            p0 = pl.multiple_of(sid * t_pix + q * PIX, PIX)
            r0 = pl.multiple_of(p0 // 128, PIX // 128)
            cp_m = pltpu.async_copy(
                im_hbm.at[b, :, pl.ds(r0, PIX // 128)], imbuf, sem_in)
            cp_w = pltpu.async_copy(
                wts_hbm.at[b, :, pl.ds(r0, PIX // 128)], wbuf, sem_in)
            cp_i = pltpu.async_copy(
                idx_hbm.at[core, b, :, pl.ds(r0, PIX // 128)], ibuf, sem_in)
            cp_m.wait()
            cp_w.wait()
            cp_i.wait()
            for g in range(PIX // 16):
                row, col = g // 8, (g % 8) * 16
                vcs = [imbuf[c, row, pl.ds(col, 16)] for c in range(8)]
                for k in range(4):
                    rows = iota + (k * PIX + g * 16)
                    wv = wbuf[k, row, pl.ds(col, 16)]
                    for c in range(8):
                        plsc.store_scatter(stage, [rows, cols[c]],
                                           vcs[c] * wv)
            for k in range(4):
                for j in range(PIX // 128):
                    pltpu.sync_copy(
                        stage.at[pl.ds(k * PIX + j * 128, 128)],
                        acc.at[ibuf.at[k, j]],
                        add=True,
                    )

        plsc.subcore_barrier()
        out0 = pl.multiple_of(core * half + sid * slc, slc)
        pltpu.sync_copy(
            acc.at[pl.ds(my0, slc)],
            out_hbm.at[b, pl.ds(out0, slc)],
        )
        plsc.subcore_barrier()


def kernel(im0, flow):
    B, C, H, W = im0.shape
    HW = H * W
    half = HW // 2
    hblk = 64
    n = hblk * W

    grid = (B, H // hblk)
    idx, wts, imr = pl.pallas_call(
        functools.partial(_prep_kernel, hblk=hblk, W=W, H=H),
        grid=grid,
        in_specs=[
            pl.BlockSpec((1, 8, hblk, W), lambda b, hb: (b, 0, hb, 0)),
            pl.BlockSpec((1, hblk, W, 2), lambda b, hb: (b, hb, 0, 0)),
        ],
        out_specs=[
            pl.BlockSpec((2, 1, 4, n // 128, 128),
                         lambda b, hb: (0, b, 0, hb, 0)),
            pl.BlockSpec((1, 4, n // 128, 128), lambda b, hb: (b, 0, hb, 0)),
            pl.BlockSpec((1, 8, n // 128, 128), lambda b, hb: (b, 0, hb, 0)),
        ],
        out_shape=[
            jax.ShapeDtypeStruct((2, B, 4, HW // 128, 128), jnp.int32),
            jax.ShapeDtypeStruct((B, 4, HW // 128, 128), jnp.float32),
            jax.ShapeDtypeStruct((B, 8, HW // 128, 128), jnp.float32),
        ],
    )(im0, flow)

    zeros = jnp.zeros((half // NS, 8), jnp.float32)

    mesh = plsc.VectorSubcoreMesh(core_axis_name="c", subcore_axis_name="s")
    acc = pl.kernel(
        functools.partial(_sc_scatter, B=B, HW=HW),
        out_type=jax.ShapeDtypeStruct((B, HW, 8), jnp.float32),
        mesh=mesh,
        compiler_params=pltpu.CompilerParams(
            use_tc_tiling_on_sc=False, needs_layout_passes=False
        ),
        scratch_types=[
            pltpu.VMEM_SHARED((half + DUMP, 8), jnp.float32),
            pltpu.VMEM((8, PIX // 128, 128), jnp.float32),
            pltpu.VMEM((4, PIX // 128, 128), jnp.float32),
            pltpu.VMEM((4, PIX // 128, 128), jnp.int32),
            pltpu.VMEM((4 * PIX, 8), jnp.float32),
            pltpu.SemaphoreType.DMA,
        ],
    )(imr, wts, idx, zeros)

    out = pl.pallas_call(
        functools.partial(_post_kernel, hblk=hblk, W=W),
        grid=grid,
        in_specs=[
            pl.BlockSpec((1, n, 8), lambda b, hb: (b, hb, 0)),
        ],
        out_specs=pl.BlockSpec((1, 8, hblk, W), lambda b, hb: (b, 0, hb, 0)),
        out_shape=jax.ShapeDtypeStruct((B, C, H, W), jnp.float32),
    )(acc)
    return out


# speedup vs baseline: 8.7221x; 1.5866x over previous
import functools

import jax
import jax.numpy as jnp
from jax import lax
from jax.experimental import pallas as pl
from jax.experimental.pallas import tpu as pltpu
from jax.experimental.pallas import tpu_sc as plsc

NC = 2
NS = 16
DUMP = 2048
PIX = 512


def _prep_kernel(im0_ref, flow_ref, idx_ref, wts_ref, imr_ref, *, hblk, W, H):
    hb = pl.program_id(1)
    half = (H // 2) * W
    nr = hblk * W // 128
    wt = W // 128
    f3 = flow_ref[0].reshape(nr, 2, 128)
    fx = f3[:, 0]
    fy = f3[:, 1]
    r_i = lax.broadcasted_iota(jnp.int32, (nr, 128), 0)
    l_i = lax.broadcasted_iota(jnp.int32, (nr, 128), 1)
    gxi = lax.rem(r_i, wt) * 128 + l_i
    gyi = r_i // wt + hb * hblk
    x = gxi.astype(jnp.float32) + fx
    y = gyi.astype(jnp.float32) + fy
    x0 = jnp.floor(x)
    y0 = jnp.floor(y)
    frx = x - x0
    fry = y - y0
    p = gyi * W + gxi
    dump = half + jnp.bitwise_and(p, DUMP - 1)

    for c in range(8):
        imr_ref[0, c] = im0_ref[0, c].reshape(nr, 128)

    k = 0
    for h in (0, 1):
        iy = y0 + h
        wy = fry if h else (1.0 - fry)
        iyi = jnp.clip(iy.astype(jnp.int32), 0, H - 1)
        yok = (iy >= 0) & (iy < H)
        for s in (0, 1):
            ix = x0 + s
            wx = frx if s else (1.0 - frx)
            ixi = jnp.clip(ix.astype(jnp.int32), 0, W - 1)
            ok = yok & (ix >= 0) & (ix < W)
            wts_ref[0, k] = jnp.where(ok, wx * wy, 0.0)
            gidx = iyi * W + ixi
            idx_ref[0, 0, k] = jnp.where(gidx < half, gidx, dump)
            idx_ref[1, 0, k] = jnp.where(gidx >= half, gidx - half, dump)
            k += 1


def _post_kernel(acc_ref, out_ref, *, hblk, W):
    m = hblk * W * 8 // 128
    a = acc_ref[0].reshape(m, 16, 8)
    t = jnp.transpose(a, (2, 0, 1)).reshape(8, m * 16)
    for c in range(8):
        out_ref[0, c] = t[c].reshape(hblk, W)


def _sc_scatter(im_hbm, wts_hbm, idx_hbm, zeros_hbm, out_hbm, acc, imbuf,
                wbuf, ibuf, stage, sem_in, *, B, HW):
    core = lax.axis_index("c")
    sid = lax.axis_index("s")
    half = HW // 2
    slc = half // NS
    t_pix = HW // NS
    nchunk = t_pix // PIX
    my0 = pl.multiple_of(sid * slc, slc)
    iota = lax.iota(jnp.int32, 16)
    cols = [jnp.full((16,), c, jnp.int32) for c in range(8)]

    @pl.loop(0, B)
    def _round(b):
        pltpu.sync_copy(zeros_hbm, acc.at[pl.ds(my0, slc)])
        plsc.subcore_barrier()

        @pl.loop(0, nchunk)
        def _chunk(q):
            p0 = pl.multiple_of(sid * t_pix + q * PIX, PIX)
            r0 = pl.multiple_of(p0 // 128, PIX // 128)
            cp_m = pltpu.async_copy(
                im_hbm.at[b, :, pl.ds(r0, PIX // 128)], imbuf, sem_in)
            cp_w = pltpu.async_copy(
                wts_hbm.at[b, :, pl.ds(r0, PIX // 128)], wbuf, sem_in)
            cp_i = pltpu.async_copy(
                idx_hbm.at[core, b, :, pl.ds(r0, PIX // 128)], ibuf, sem_in)
            cp_m.wait()
            cp_w.wait()
            cp_i.wait()
            for g in range(PIX // 16):
                row, col = g // 8, (g % 8) * 16
                vcs = [imbuf[c, row, pl.ds(col, 16)] for c in range(8)]
                for k in range(4):
                    rows = iota + (k * PIX + g * 16)
                    wv = wbuf[k, row, pl.ds(col, 16)]
                    for c in range(8):
                        plsc.store_scatter(stage, [rows, cols[c]],
                                           vcs[c] * wv)
            for k in range(4):
                for j in range(PIX // 128):
                    pltpu.sync_copy(
                        stage.at[pl.ds(k * PIX + j * 128, 128)],
                        acc.at[ibuf.at[k, j]],
                        add=True,
                    )

        plsc.subcore_barrier()
        out0 = pl.multiple_of(core * half + sid * slc, slc)
        pltpu.sync_copy(
            acc.at[pl.ds(my0, slc)],
            out_hbm.at[b, pl.ds(out0, slc)],
        )
        plsc.subcore_barrier()


def kernel(im0, flow):
    B, C, H, W = im0.shape
    HW = H * W
    half = HW // 2
    hblk = 64
    n = hblk * W

    grid = (B, H // hblk)
    flowv = jnp.transpose(flow.reshape(B, H, W // 128, 128, 2),
                          (0, 1, 2, 4, 3)).reshape(B, H * (W // 128) * 2, 128)
    idx, wts, imr = pl.pallas_call(
        functools.partial(_prep_kernel, hblk=hblk, W=W, H=H),
        grid=grid,
        in_specs=[
            pl.BlockSpec((1, 8, hblk, W), lambda b, hb: (b, 0, hb, 0)),
            pl.BlockSpec((1, hblk * (W // 128) * 2, 128),
                         lambda b, hb: (b, hb, 0)),
        ],
        out_specs=[
            pl.BlockSpec((2, 1, 4, n // 128, 128),
                         lambda b, hb: (0, b, 0, hb, 0)),
            pl.BlockSpec((1, 4, n // 128, 128), lambda b, hb: (b, 0, hb, 0)),
            pl.BlockSpec((1, 8, n // 128, 128), lambda b, hb: (b, 0, hb, 0)),
        ],
        out_shape=[
            jax.ShapeDtypeStruct((2, B, 4, HW // 128, 128), jnp.int32),
            jax.ShapeDtypeStruct((B, 4, HW // 128, 128), jnp.float32),
            jax.ShapeDtypeStruct((B, 8, HW // 128, 128), jnp.float32),
        ],
    )(im0, flowv)

    zeros = jnp.zeros((half // NS, 8), jnp.float32)

    mesh = plsc.VectorSubcoreMesh(core_axis_name="c", subcore_axis_name="s")
    acc = pl.kernel(
        functools.partial(_sc_scatter, B=B, HW=HW),
        out_type=jax.ShapeDtypeStruct((B, HW, 8), jnp.float32),
        mesh=mesh,
        compiler_params=pltpu.CompilerParams(
            use_tc_tiling_on_sc=False, needs_layout_passes=False
        ),
        scratch_types=[
            pltpu.VMEM_SHARED((half + DUMP, 8), jnp.float32),
            pltpu.VMEM((8, PIX // 128, 128), jnp.float32),
            pltpu.VMEM((4, PIX // 128, 128), jnp.float32),
            pltpu.VMEM((4, PIX // 128, 128), jnp.int32),
            pltpu.VMEM((4 * PIX, 8), jnp.float32),
            pltpu.SemaphoreType.DMA,
        ],
    )(imr, wts, idx, zeros)

    accv = acc.reshape(B, HW * 8 // 128, 128)
    out = pl.pallas_call(
        functools.partial(_post_kernel, hblk=hblk, W=W),
        grid=grid,
        in_specs=[
            pl.BlockSpec((1, n * 8 // 128, 128), lambda b, hb: (b, hb, 0)),
        ],
        out_specs=pl.BlockSpec((1, 8, hblk, W), lambda b, hb: (b, 0, hb, 0)),
        out_shape=jax.ShapeDtypeStruct((B, C, H, W), jnp.float32),
    )(accv)
    return out
